# baseline (speedup 1.0000x reference)
import jax
import jax.numpy as jnp
from jax import lax
from jax.experimental import pallas as pl
from jax.experimental.pallas import tpu as pltpu
from jax.experimental.pallas import tpu_sc as plsc

_BATCH = 16384
_SEQ = 100
_OUT_DIM = 128
_ROWS = _SEQ * _BATCH
_NW = 32
_PER_W = _ROWS // _NW
_CHUNK = 512
_NCOPY = _PER_W // _CHUNK
_L = 16


def _sc_fill(emb_hbm, out_hbm, scal_v, buf_v):
    c = lax.axis_index("c")
    s = lax.axis_index("s")
    wid = s * 2 + c

    pltpu.sync_copy(emb_hbm, scal_v)
    v = scal_v[...]

    def fill(r, carry):
        for k in range(_OUT_DIM // _L):
            buf_v[r, pl.ds(k * _L, _L)] = v
        return carry

    lax.fori_loop(0, _CHUNK, fill, 0)

    base = wid * _PER_W

    def copy(i, carry):
        pltpu.sync_copy(buf_v, out_hbm.at[pl.ds(base + i * _CHUNK, _CHUNK)])
        return carry

    lax.fori_loop(0, _NCOPY, copy, 0)


def kernel(inputs, emb_table):
    del inputs
    emb_flat = jnp.broadcast_to(emb_table.reshape((1,)), (_L,))
    out = pl.kernel(
        _sc_fill,
        out_type=jax.ShapeDtypeStruct((_ROWS, _OUT_DIM), jnp.float32),
        mesh=plsc.VectorSubcoreMesh(core_axis_name="c", subcore_axis_name="s"),
        scratch_types=[
            pltpu.VMEM((_L,), jnp.float32),
            pltpu.VMEM((_CHUNK, _OUT_DIM), jnp.float32),
        ],
    )(emb_flat)
    return jnp.transpose(out.reshape(_SEQ, _BATCH, _OUT_DIM), (1, 0, 2))
